# baseline (speedup 1.0000x reference)
import functools

import jax
import jax.numpy as jnp
from jax.experimental import pallas as pl
from jax.experimental.pallas import tpu as pltpu
from jax._src.pallas.mosaic import sc_core as plsc
from jax._src.pallas.mosaic import sc_primitives as plscp

_NUM_TYPES = 10
_E = 32
_K = 16
_OVER = 5.0
_TL = 2048
_B_SC = 4
_N_SUBCORES = 32

_MU = (0.10000000149011612, 0.20000000298023224, 0.30000001192092896,
       0.4000000059604645, 0.5, 0.6000000238418579, 0.699999988079071,
       0.800000011920929, 0.8999999761581421, 1.0)


def _rotl(x, r):
    return (x << jnp.uint32(r)) | (x >> jnp.uint32(32 - r))


def _threefry_bits(k1_int, x1):
    k1i = k1_int & 0xFFFFFFFF
    ks2i = (0x1BD11BDA ^ k1i) & 0xFFFFFFFF
    x0 = jnp.zeros_like(x1)
    x1 = x1 + jnp.uint32(k1i)
    rots0 = (13, 15, 26, 6)
    rots1 = (17, 29, 16, 24)
    inj = ((k1i, ks2i + 1), (ks2i, 2), (0, k1i + 3), (k1i, ks2i + 4),
           (ks2i, 5))
    for g in range(5):
        for r in (rots0 if g % 2 == 0 else rots1):
            x0 = x0 + x1
            x1 = _rotl(x1, r)
            x1 = x1 ^ x0
        a, bb = inj[g]
        if a:
            x0 = x0 + jnp.uint32(a)
        x1 = x1 + jnp.uint32(bb & 0xFFFFFFFF)
    return x0 ^ x1


def _bits_to_uniform(bits):
    f = jax.lax.bitcast_convert_type(
        (bits >> jnp.uint32(9)) | jnp.uint32(0x3F800000), jnp.float32)
    return f - jnp.float32(1.0)


def _row_stats(t, dt, ty):
    te = jnp.zeros_like(t)
    for k in range(_NUM_TYPES):
        te = te + jnp.where(ty == k, jnp.float32(_MU[k]), jnp.float32(0.0))
    base = jnp.float32(0.1) + jax.nn.softplus(
        te + jnp.float32(0.1) * dt + jnp.float32(0.01) * jnp.cos(t))
    v0 = jnp.zeros_like(base)
    for k in range(_NUM_TYPES):
        v0 = v0 + (base * jnp.float32(_MU[k]) + jnp.float32(0.05))
    M = v0 * jnp.float32(_OVER)
    return base, M


def _expj_thr(b, l0, t, dt, ty, W, L):
    base, M = _row_stats(t, dt, ty)
    sub = jax.lax.broadcasted_iota(jnp.int32, (_E, W), 0)
    lane = jax.lax.broadcasted_iota(jnp.int32, (_E, W), 1)
    ie = (b * (L * _E) + (l0 + lane) * _E + sub).astype(jnp.uint32)
    u1 = _bits_to_uniform(_threefry_bits(1, ie))
    e = -jnp.log1p(-u1)
    x = e / M
    for s in (1, 2, 4, 8, 16):
        shifted = jnp.concatenate(
            [jnp.zeros((s, W), jnp.float32), x[:-s, :]], axis=0)
        x = x + shifted
    exp_j = x
    st = base * jnp.exp(jnp.float32(-0.5) * exp_j)
    intens = jnp.zeros_like(st)
    for k in range(_NUM_TYPES):
        intens = intens + (st * jnp.float32(_MU[k]) + jnp.float32(0.05))
    thr = (intens / M) * jnp.float32(8388608.0)
    return exp_j, thr, lane, sub


def _tc_body(t_ref, dt_ref, ty_ref, out_ref, *, L):
    b = pl.program_id(0)
    lt = pl.program_id(1)
    l0 = lt * _TL
    exp_j, thr, lane, sub = _expj_thr(
        b, l0, t_ref[0], dt_ref[0], ty_ref[0], _TL, L)
    rows = []
    big = jnp.float32(jnp.inf)
    iu0 = b * (L * _K * _E) + (l0 + lane) * (_K * _E) + sub
    for k in range(_K):
        iu = (iu0 + k * _E).astype(jnp.uint32)
        mant = _threefry_bits(2, iu) >> jnp.uint32(9)
        mf = mant.astype(jnp.int32).astype(jnp.float32)
        cand = jnp.where(mf < thr, exp_j, big)
        mval = jnp.min(cand, axis=0, keepdims=True)
        res = jnp.where(mval == big, jnp.float32(0.0),
                        jnp.minimum(mval, jnp.float32(100000.0)))
        rows.append(res)
    out_ref[...] = jnp.concatenate(rows, axis=0)


def _prep_body(t_ref, dt_ref, ty_ref, ej_ref, it_ref, *, L, R0):
    b = pl.program_id(0) + R0
    exp_j, thr, _, _ = _expj_thr(
        b, 0, t_ref[0], dt_ref[0], ty_ref[0], L, L)
    ej_ref[...] = exp_j
    it_ref[...] = jnp.ceil(thr).astype(jnp.int32)


def _sc_body(ej_hbm, it_hbm, out_hbm, ev, tv, ov, *, L, F0, FLAT0, CH):
    wid = jax.lax.axis_index("c") * 16 + jax.lax.axis_index("s")
    fw = F0 + wid * CH
    c0 = fw - FLAT0
    pltpu.sync_copy(ej_hbm.at[:, pl.ds(c0, CH)], ev)
    pltpu.sync_copy(it_hbm.at[:, pl.ds(c0, CH)], tv)
    lanes = jax.lax.iota(jnp.int32, 16)
    big = jnp.full((16,), jnp.inf, jnp.float32)
    zero = jnp.zeros((16,), jnp.float32)
    cap = jnp.full((16,), 100000.0, jnp.float32)

    def step(i, carry):
        lg = i // _K
        k = i % _K
        f = fw + lg * 16
        bg = f // L
        l = f - bg * L
        cbase = bg * (L * _K * _E) + (l + lanes) * (_K * _E) + k * _E
        macc = big
        for j in range(_E):
            bits = _threefry_bits(2, (cbase + j).astype(jnp.uint32))
            mant = bits >> jnp.uint32(9)
            tvec = tv[j, pl.ds(lg * 16, 16)]
            evec = ev[j, pl.ds(lg * 16, 16)]
            ok = mant < plscp.bitcast(tvec, jnp.uint32)
            macc = jnp.minimum(macc, jnp.where(ok, evec, big))
        res = jnp.where(macc == big, zero, jnp.minimum(macc, cap))
        ov[k, pl.ds(lg * 16, 16)] = res
        return carry

    jax.lax.fori_loop(0, (CH // 16) * _K, step, 0)
    pltpu.sync_copy(ov, out_hbm.at[:, pl.ds(c0, CH)])


def kernel(time_seqs, time_delta_seqs, type_seqs, num_sample):
    B, L = time_seqs.shape
    r0 = B - _B_SC
    F0 = r0 * L
    CH = _B_SC * L // _N_SUBCORES
    t3 = time_seqs.reshape(B, 1, L)
    dt3 = time_delta_seqs.reshape(B, 1, L)
    ty3 = type_seqs.reshape(B, 1, L)

    prep_in = pl.BlockSpec((1, 1, L), lambda i: (i + r0, 0, 0))
    ej, it = pl.pallas_call(
        functools.partial(_prep_body, L=L, R0=r0),
        grid=(_B_SC,),
        in_specs=[prep_in, prep_in, prep_in],
        out_specs=[pl.BlockSpec((_E, L), lambda i: (0, i)),
                   pl.BlockSpec((_E, L), lambda i: (0, i))],
        out_shape=[jax.ShapeDtypeStruct((_E, _B_SC * L), jnp.float32),
                   jax.ShapeDtypeStruct((_E, _B_SC * L), jnp.int32)],
    )(t3, dt3, ty3)

    sc_fn = pl.kernel(
        functools.partial(_sc_body, L=L, F0=F0, FLAT0=F0, CH=CH),
        out_type=jax.ShapeDtypeStruct((_K, _B_SC * L), jnp.float32),
        mesh=plsc.VectorSubcoreMesh(core_axis_name="c",
                                    subcore_axis_name="s"),
        scratch_types=[pltpu.VMEM((_E, CH), jnp.float32),
                       pltpu.VMEM((_E, CH), jnp.int32),
                       pltpu.VMEM((_K, CH), jnp.float32)],
    )
    out_sc = sc_fn(ej, it)

    in_spec = pl.BlockSpec((1, 1, _TL), lambda b, lt: (b, 0, lt))
    out_tc = pl.pallas_call(
        functools.partial(_tc_body, L=L),
        grid=(r0, L // _TL),
        in_specs=[in_spec, in_spec, in_spec],
        out_specs=pl.BlockSpec((_K, _TL), lambda b, lt: (b, lt)),
        out_shape=jax.ShapeDtypeStruct((r0 * _K, L), jnp.float32),
        compiler_params=pltpu.CompilerParams(
            dimension_semantics=("parallel", "parallel")),
    )(t3[:r0], dt3[:r0], ty3[:r0])

    res = jnp.concatenate(
        [out_tc.reshape(r0, _K, L).transpose(0, 2, 1),
         out_sc.reshape(_K, _B_SC, L).transpose(1, 2, 0)], axis=0)
    weights = jnp.ones((B, L, _K), jnp.float32) / num_sample
    return (res, weights)


# speedup vs baseline: 1.0041x; 1.0041x over previous
import functools

import jax
import jax.numpy as jnp
from jax.experimental import pallas as pl
from jax.experimental.pallas import tpu as pltpu
from jax._src.pallas.mosaic import sc_core as plsc
from jax._src.pallas.mosaic import sc_primitives as plscp

_NUM_TYPES = 10
_E = 32
_K = 16
_OVER = 5.0
_TL = 2048
_B_SC = 4
_N_SUBCORES = 32

_MU = (0.10000000149011612, 0.20000000298023224, 0.30000001192092896,
       0.4000000059604645, 0.5, 0.6000000238418579, 0.699999988079071,
       0.800000011920929, 0.8999999761581421, 1.0)


def _rotl(x, r):
    return (x << jnp.uint32(r)) | (x >> jnp.uint32(32 - r))


def _threefry_bits(k1_int, x1):
    k1i = k1_int & 0xFFFFFFFF
    ks2i = (0x1BD11BDA ^ k1i) & 0xFFFFFFFF
    x0 = jnp.zeros_like(x1)
    x1 = x1 + jnp.uint32(k1i)
    rots0 = (13, 15, 26, 6)
    rots1 = (17, 29, 16, 24)
    inj = ((k1i, ks2i + 1), (ks2i, 2), (0, k1i + 3), (k1i, ks2i + 4),
           (ks2i, 5))
    for g in range(5):
        for r in (rots0 if g % 2 == 0 else rots1):
            x0 = x0 + x1
            x1 = _rotl(x1, r)
            x1 = x1 ^ x0
        a, bb = inj[g]
        if a:
            x0 = x0 + jnp.uint32(a)
        x1 = x1 + jnp.uint32(bb & 0xFFFFFFFF)
    return x0 ^ x1


def _bits_to_uniform(bits):
    f = jax.lax.bitcast_convert_type(
        (bits >> jnp.uint32(9)) | jnp.uint32(0x3F800000), jnp.float32)
    return f - jnp.float32(1.0)


def _row_stats(t, dt, ty):
    te = jnp.zeros_like(t)
    for k in range(_NUM_TYPES):
        te = te + jnp.where(ty == k, jnp.float32(_MU[k]), jnp.float32(0.0))
    base = jnp.float32(0.1) + jax.nn.softplus(
        te + jnp.float32(0.1) * dt + jnp.float32(0.01) * jnp.cos(t))
    v0 = jnp.zeros_like(base)
    for k in range(_NUM_TYPES):
        v0 = v0 + (base * jnp.float32(_MU[k]) + jnp.float32(0.05))
    M = v0 * jnp.float32(_OVER)
    return base, M


def _expj_thr(b, l0, t, dt, ty, W, L):
    base, M = _row_stats(t, dt, ty)
    sub = jax.lax.broadcasted_iota(jnp.int32, (_E, W), 0)
    lane = jax.lax.broadcasted_iota(jnp.int32, (_E, W), 1)
    ie = (b * (L * _E) + (l0 + lane) * _E + sub).astype(jnp.uint32)
    u1 = _bits_to_uniform(_threefry_bits(1, ie))
    e = -jnp.log1p(-u1)
    x = e / M
    for s in (1, 2, 4, 8, 16):
        shifted = jnp.concatenate(
            [jnp.zeros((s, W), jnp.float32), x[:-s, :]], axis=0)
        x = x + shifted
    exp_j = x
    st = base * jnp.exp(jnp.float32(-0.5) * exp_j)
    intens = jnp.zeros_like(st)
    for k in range(_NUM_TYPES):
        intens = intens + (st * jnp.float32(_MU[k]) + jnp.float32(0.05))
    thr = (intens / M) * jnp.float32(8388608.0)
    return exp_j, thr, lane, sub


def _one_sample(k, iu0, thr, exp_j):
    big = jnp.float32(jnp.inf)
    iu = (iu0 + k * _E).astype(jnp.uint32)
    mant = _threefry_bits(2, iu) >> jnp.uint32(9)
    mf = mant.astype(jnp.int32).astype(jnp.float32)
    cand = jnp.where(mf < thr, exp_j, big)
    mval = jnp.min(cand, axis=0, keepdims=True)
    return jnp.where(mval == big, jnp.float32(0.0),
                     jnp.minimum(mval, jnp.float32(100000.0)))


def _tc_body(t_ref, dt_ref, ty_ref, out_ref, *, L, R_SPLIT):
    b = pl.program_id(0)
    lt = pl.program_id(1)
    l0 = lt * _TL
    exp_j, thr, lane, sub = _expj_thr(
        b, l0, t_ref[0], dt_ref[0], ty_ref[0], _TL, L)
    iu0 = b * (L * _K * _E) + (l0 + lane) * (_K * _E) + sub
    out_ref[0:_K // 2, :] = jnp.concatenate(
        [_one_sample(k, iu0, thr, exp_j) for k in range(_K // 2)], axis=0)

    @pl.when(b < R_SPLIT)
    def _():
        out_ref[_K // 2:, :] = jnp.concatenate(
            [_one_sample(k, iu0, thr, exp_j)
             for k in range(_K // 2, _K)], axis=0)


def _prep_body(t_ref, dt_ref, ty_ref, ej_ref, it_ref, *, L, R0):
    b = pl.program_id(0) + R0
    exp_j, thr, _, _ = _expj_thr(
        b, 0, t_ref[0], dt_ref[0], ty_ref[0], L, L)
    ej_ref[...] = exp_j
    it_ref[...] = jnp.ceil(thr).astype(jnp.int32)


def _sc_minloop(cbase, tv, ev, coff, big):
    macc = big
    for j in range(_E):
        bits = _threefry_bits(2, (cbase + j).astype(jnp.uint32))
        mant = bits >> jnp.uint32(9)
        tvec = tv[j, pl.ds(coff, 16)]
        evec = ev[j, pl.ds(coff, 16)]
        ok = mant < plscp.bitcast(tvec, jnp.uint32)
        macc = jnp.minimum(macc, jnp.where(ok, evec, big))
    return macc


def _sc_body(ej_hbm, it_hbm, out_hbm, out2_hbm, ev, tv, ov, ev2, tv2, ov2,
             *, L, F0, FLAT0, CH, R_SPLIT, XW):
    wid = jax.lax.axis_index("c") * 16 + jax.lax.axis_index("s")
    fw = F0 + wid * CH
    c0 = fw - FLAT0
    pltpu.sync_copy(ej_hbm.at[:, pl.ds(c0, CH)], ev)
    pltpu.sync_copy(it_hbm.at[:, pl.ds(c0, CH)], tv)
    kk = _K // 2 + wid % (_K // 2)
    cb = wid // (_K // 2) * XW
    pltpu.sync_copy(ej_hbm.at[:, pl.ds(cb, XW)], ev2)
    pltpu.sync_copy(it_hbm.at[:, pl.ds(cb, XW)], tv2)
    lanes = jax.lax.iota(jnp.int32, 16)
    big = jnp.full((16,), jnp.inf, jnp.float32)
    zero = jnp.zeros((16,), jnp.float32)
    cap = jnp.full((16,), 100000.0, jnp.float32)

    def step(i, carry):
        lg = i // _K
        k = i % _K
        f = fw + lg * 16
        bg = f // L
        l = f - bg * L
        cbase = bg * (L * _K * _E) + (l + lanes) * (_K * _E) + k * _E
        macc = _sc_minloop(cbase, tv, ev, lg * 16, big)
        ov[k, pl.ds(lg * 16, 16)] = jnp.where(
            macc == big, zero, jnp.minimum(macc, cap))
        return carry

    jax.lax.fori_loop(0, (CH // 16) * _K, step, 0)
    pltpu.sync_copy(ov, out_hbm.at[:, pl.ds(c0, CH)])

    def step2(i, carry):
        l = cb + i * 16
        cbase = (R_SPLIT * (L * _K * _E) + (l + lanes) * (_K * _E)
                 + kk * _E)
        macc = _sc_minloop(cbase, tv2, ev2, i * 16, big)
        ov2[pl.ds(i * 16, 16)] = jnp.where(
            macc == big, zero, jnp.minimum(macc, cap))
        return carry

    jax.lax.fori_loop(0, XW // 16, step2, 0)
    pltpu.sync_copy(ov2, out2_hbm.at[pl.ds(wid * XW, XW)])


def kernel(time_seqs, time_delta_seqs, type_seqs, num_sample):
    B, L = time_seqs.shape
    r_split = B - _B_SC - 1
    n_tc = r_split + 1
    n_prep = _B_SC + 1
    F0 = (r_split + 1) * L
    FLAT0 = r_split * L
    CH = _B_SC * L // _N_SUBCORES
    XW = L // (_N_SUBCORES // (_K // 2))
    t3 = time_seqs.reshape(B, 1, L)
    dt3 = time_delta_seqs.reshape(B, 1, L)
    ty3 = type_seqs.reshape(B, 1, L)

    prep_in = pl.BlockSpec((1, 1, L), lambda i: (i + r_split, 0, 0))
    ej, it = pl.pallas_call(
        functools.partial(_prep_body, L=L, R0=r_split),
        grid=(n_prep,),
        in_specs=[prep_in, prep_in, prep_in],
        out_specs=[pl.BlockSpec((_E, L), lambda i: (0, i)),
                   pl.BlockSpec((_E, L), lambda i: (0, i))],
        out_shape=[jax.ShapeDtypeStruct((_E, n_prep * L), jnp.float32),
                   jax.ShapeDtypeStruct((_E, n_prep * L), jnp.int32)],
    )(t3, dt3, ty3)

    sc_fn = pl.kernel(
        functools.partial(_sc_body, L=L, F0=F0, FLAT0=FLAT0, CH=CH,
                          R_SPLIT=r_split, XW=XW),
        out_type=[jax.ShapeDtypeStruct((_K, _B_SC * L), jnp.float32),
                  jax.ShapeDtypeStruct((_N_SUBCORES * XW,), jnp.float32)],
        mesh=plsc.VectorSubcoreMesh(core_axis_name="c",
                                    subcore_axis_name="s"),
        scratch_types=[pltpu.VMEM((_E, CH), jnp.float32),
                       pltpu.VMEM((_E, CH), jnp.int32),
                       pltpu.VMEM((_K, CH), jnp.float32),
                       pltpu.VMEM((_E, XW), jnp.float32),
                       pltpu.VMEM((_E, XW), jnp.int32),
                       pltpu.VMEM((XW,), jnp.float32)],
    )
    out_sc, out_sc2 = sc_fn(ej, it)

    in_spec = pl.BlockSpec((1, 1, _TL), lambda b, lt: (b, 0, lt))
    out_tc = pl.pallas_call(
        functools.partial(_tc_body, L=L, R_SPLIT=r_split),
        grid=(n_tc, L // _TL),
        in_specs=[in_spec, in_spec, in_spec],
        out_specs=pl.BlockSpec((_K, _TL), lambda b, lt: (b, lt)),
        out_shape=jax.ShapeDtypeStruct((n_tc * _K, L), jnp.float32),
        compiler_params=pltpu.CompilerParams(
            dimension_semantics=("parallel", "parallel")),
    )(t3[:n_tc], dt3[:n_tc], ty3[:n_tc])

    res_tc = out_tc.reshape(n_tc, _K, L).transpose(0, 2, 1)
    sc_half = (out_sc2.reshape(_N_SUBCORES // (_K // 2), _K // 2, XW)
               .transpose(0, 2, 1).reshape(L, _K // 2))
    row_split = jnp.concatenate([res_tc[r_split, :, :_K // 2], sc_half],
                                axis=1)
    res = jnp.concatenate(
        [res_tc[:r_split], row_split[None],
         out_sc.reshape(_K, _B_SC, L).transpose(1, 2, 0)], axis=0)
    weights = jnp.ones((B, L, _K), jnp.float32) / num_sample
    return (res, weights)
